# baseline (speedup 1.0000x reference)
import functools

import jax
import jax.numpy as jnp
from jax import lax
from jax.experimental import pallas as pl
from jax.experimental.pallas import tpu as pltpu
from jax.experimental.pallas import tpu_sc as plsc

_B = 16384
_K = 16
_NC = 2
_NS = 16
_NW = _NC * _NS
_L = 16
_CH_R = 1024
_NB = 32
_CAP = 64
_SLOTS = _NB * _CAP
_ROW_LO = 262144
_FULL_END = 999424
_TAIL0 = 999936
_NU = 1000000


def _wid():
    return lax.axis_index("s") * _NC + lax.axis_index("c")


def _popcnt(m):
    pc = plsc.all_reduce_population_count(m)
    return lax.squeeze(lax.slice(pc, (0,), (1,)), (0,))


def _k1_body(xu_hbm, xv_hbm, wt_hbm, ht_hbm, wtail_hbm, htail_hbm,
             uflat_hbm, vflat_hbm, uposl_hbm, vposl_hbm,
             xlu, xlv, wlu_val, wlu_pos, wlv_val, wlv_pos,
             bu_u, bu_p, bv_u, bv_p, cnts_u, cnts_v,
             sw0, sw1, sh0, sh1, ew, eh, tw, th, st, stu1, stv0, stv1,
             semw0, semw1, semh0, semh1, semou, semov):
    wid = _wid()
    iota = lax.iota(jnp.int32, _L)
    lo = jnp.where(wid < 8, wid * 32768, _ROW_LO + (wid - 8) * 30720)
    nch = jnp.where(wid < 8, 32, 30)
    hi = jnp.where(wid == _NW - 1, _NU, lo + nch * _CH_R)
    lo_off = pl.multiple_of(lo, 128)

    pltpu.async_copy(wt_hbm.at[:, pl.ds(lo_off, _CH_R)], sw0, semw0)
    pltpu.async_copy(ht_hbm.at[:, pl.ds(lo_off, _CH_R)], sh0, semh0)
    off1 = pl.multiple_of(lo + _CH_R, 128)
    pltpu.async_copy(wt_hbm.at[:, pl.ds(off1, _CH_R)], sw1, semw1)
    pltpu.async_copy(ht_hbm.at[:, pl.ds(off1, _CH_R)], sh1, semh1)

    @pl.when(wid == _NW - 1)
    def _():
        pltpu.sync_copy(wt_hbm.at[:, pl.ds(_FULL_END, 512)], ew)
        pltpu.sync_copy(ht_hbm.at[:, pl.ds(_FULL_END, 512)], eh)
        pltpu.sync_copy(wtail_hbm, tw)
        pltpu.sync_copy(htail_hbm, th)

    neg1 = jnp.full((_L,), -1, jnp.int32)

    def init(i, _):
        bu_p[pl.ds(i * _L, _L)] = neg1
        bv_p[pl.ds(i * _L, _L)] = neg1
        return 0

    lax.fori_loop(0, _SLOTS // _L, init, 0)

    half_n = _B // 2

    def scan_half(p, carry):
        pltpu.sync_copy(xu_hbm.at[pl.ds(p * half_n, half_n)], xlu)
        pltpu.sync_copy(xv_hbm.at[pl.ds(p * half_n, half_n)], xlv)
        posb = p * half_n

        def scan(t, c2):
            cu, cv = c2
            u = xlu[pl.ds(t * _L, _L)]
            v = xlv[pl.ds(t * _L, _L)]
            mu = (u >= lo) & (u < hi)
            mv = (v >= lo) & (v < hi)
            pos = jnp.full((_L,), posb + t * _L, jnp.int32) + iota
            plsc.store_compressed(wlu_val.at[pl.ds(cu, _L)], u, mask=mu)
            plsc.store_compressed(wlu_pos.at[pl.ds(cu, _L)], pos, mask=mu)
            plsc.store_compressed(wlv_val.at[pl.ds(cv, _L)], v, mask=mv)
            plsc.store_compressed(wlv_pos.at[pl.ds(cv, _L)], pos, mask=mv)
            return (cu + _popcnt(mu), cv + _popcnt(mv))

        return lax.fori_loop(0, half_n // _L, scan, carry)

    nu, nv = lax.fori_loop(0, 2, scan_half, (jnp.int32(0), jnp.int32(0)))

    def distribute(n, wl_val, wl_pos, wb_u, wb_p, cnts_ref):
        def dist(q, cnts):
            val = wl_val[pl.ds(q * _L, _L)]
            pos = wl_pos[pl.ds(q * _L, _L)]
            valid = (jnp.full((_L,), q * _L, jnp.int32) + iota) < n
            bkt = lax.shift_right_logical(val - lo, 10)
            new = []
            for b in range(_NB):
                m = (bkt == b) & valid
                cb = cnts[b]
                plsc.store_compressed(
                    wb_u.at[pl.ds(b * _CAP + cb, _L)], val, mask=m)
                plsc.store_compressed(
                    wb_p.at[pl.ds(b * _CAP + cb, _L)], pos, mask=m)
                new.append(cb + _popcnt(m))
            return tuple(new)

        cnts = lax.fori_loop(0, (n + _L - 1) // _L, dist,
                             tuple(jnp.int32(0) for _ in range(_NB)))
        for half in range(2):
            acc = jnp.zeros((_L,), jnp.int32)
            for j in range(_L):
                acc = jnp.where(iota == j,
                                jnp.full((_L,), cnts[half * _L + j]), acc)
            cnts_ref[pl.ds(half * _L, _L)] = acc

    distribute(nu, wlu_val, wlu_pos, bu_u, bu_p, cnts_u)
    distribute(nv, wlv_val, wlv_pos, bv_u, bv_p, cnts_v)
    pltpu.sync_copy(bu_p, uposl_hbm.at[pl.ds(wid * _SLOTS, _SLOTS)])
    pltpu.sync_copy(bv_p, vposl_hbm.at[pl.ds(wid * _SLOTS, _SLOTS)])

    def gather_bucket(c, wb_u, cnts_ref, buf, stb, dst_hbm, sem_out, drain):
        cntc = jnp.max(plsc.load_gather(
            cnts_ref, [jnp.full((_L,), c, jnp.int32)]))
        cbase = lo + c * _CH_R

        @pl.when(drain)
        def _():
            pltpu.make_async_copy(
                stb, dst_hbm.at[pl.ds(0, _CAP * _L)], sem_out).wait()

        def grow(j, _):
            uj = plsc.load_gather(
                wb_u, [jnp.full((_L,), c * _CAP + j, jnp.int32)])
            ul = (uj - cbase) & (_CH_R - 1)
            stb[pl.ds(j * _L, _L)] = plsc.load_gather(buf, [iota, ul])
            return 0

        lax.fori_loop(0, cntc, grow, 0)
        dst = pl.multiple_of(wid * _SLOTS * _L + c * _CAP * _L, 512)
        pltpu.async_copy(stb, dst_hbm.at[pl.ds(dst, _CAP * _L)], sem_out)

    swb = (sw0, sw1)
    shb = (sh0, sh1)
    semw = (semw0, semw1)
    semh = (semh0, semh1)
    stu = (st, stu1)
    stv = (stv0, stv1)

    def pair(qp, _):
        for par in range(2):
            c = qp * 2 + par

            @pl.when(c < nch)
            def _():
                pltpu.make_async_copy(
                    wt_hbm.at[:, pl.ds(0, _CH_R)], swb[par], semw[par]).wait()
                pltpu.make_async_copy(
                    ht_hbm.at[:, pl.ds(0, _CH_R)], shb[par], semh[par]).wait()
                gather_bucket(c, bu_u, cnts_u, swb[par], stu[par],
                              uflat_hbm, semou, c >= 2)
                gather_bucket(c, bv_u, cnts_v, shb[par], stv[par],
                              vflat_hbm, semov, c >= 2)

                @pl.when(c + 2 < nch)
                def _():
                    off = pl.multiple_of(lo + (c + 2) * _CH_R, 128)
                    pltpu.async_copy(
                        wt_hbm.at[:, pl.ds(off, _CH_R)], swb[par], semw[par])
                    pltpu.async_copy(
                        ht_hbm.at[:, pl.ds(off, _CH_R)], shb[par], semh[par])
        return 0

    lax.fori_loop(0, _NB // 2, pair, 0)

    for sem_out, dst_hbm in ((semou, uflat_hbm), (semov, vflat_hbm)):
        for _ in range(2):
            pltpu.make_async_copy(
                stu[0], dst_hbm.at[pl.ds(0, _CAP * _L)], sem_out).wait()

    @pl.when(wid == _NW - 1)
    def _():
        c = 30

        def tail_bucket(wb_u, cnts_ref, ebuf, tbuf, dst_hbm):
            cntc = jnp.max(plsc.load_gather(
                cnts_ref, [jnp.full((_L,), c, jnp.int32)]))

            def grow(j, _):
                uj = plsc.load_gather(
                    wb_u, [jnp.full((_L,), c * _CAP + j, jnp.int32)])
                ul = uj - _FULL_END
                in_slab = ul < 512
                row_a = plsc.load_gather(ebuf, [iota, ul & 511])
                row_b = plsc.load_gather(
                    tbuf, [((ul - 512) & 63) * _L + iota])
                st[pl.ds(j * _L, _L)] = jnp.where(in_slab, row_a, row_b)
                return 0

            lax.fori_loop(0, cntc, grow, 0)
            dst = pl.multiple_of(wid * _SLOTS * _L + c * _CAP * _L, 512)
            pltpu.sync_copy(st, dst_hbm.at[pl.ds(dst, _CAP * _L)])

        tail_bucket(bu_u, cnts_u, ew, tw, uflat_hbm)
        tail_bucket(bv_u, cnts_v, eh, th, vflat_hbm)


def _k2_body(uflat_hbm, vflat_hbm, uposl_hbm, vposl_hbm,
             usort_hbm, vsort_hbm, rstage_u, rstage_v, idx2_u, idx2_v, sem):
    wid = _wid()
    base = wid * _SLOTS
    nchunk = _SLOTS // 128
    stages = [
        pltpu.async_copy(uflat_hbm.at[pl.ds(base, _SLOTS), :], rstage_u, sem),
        pltpu.async_copy(vflat_hbm.at[pl.ds(base, _SLOTS), :], rstage_v, sem),
    ]
    for c in range(nchunk):
        stages.append(pltpu.async_copy(
            uposl_hbm.at[pl.ds(base + c * 128, 128)], idx2_u.at[c], sem))
        stages.append(pltpu.async_copy(
            vposl_hbm.at[pl.ds(base + c * 128, 128)], idx2_v.at[c], sem))
    for cp in stages:
        cp.wait()
    copies = []
    for c in range(nchunk):
        copies.append(pltpu.async_copy(
            rstage_u.at[pl.ds(c * 128, 128), :],
            usort_hbm.at[plsc.Indices(idx2_u.at[c], ignored_value=-1)],
            sem))
        copies.append(pltpu.async_copy(
            rstage_v.at[pl.ds(c * 128, 128), :],
            vsort_hbm.at[plsc.Indices(idx2_v.at[c], ignored_value=-1)],
            sem))
    for cp in copies:
        cp.wait()


def _k3_body(usort_hbm, vsort_hbm, out_hbm, urows, vrows, outv):
    wid = _wid()
    base = wid * (_B // _NW)
    iota = lax.iota(jnp.int32, _L)
    pltpu.sync_copy(usort_hbm.at[pl.ds(base, _B // _NW), :], urows)
    pltpu.sync_copy(vsort_hbm.at[pl.ds(base, _B // _NW), :], vrows)

    def group(g, _):
        row = jnp.full((_L,), g * _L, jnp.int32) + iota
        acc = jnp.zeros((_L,), jnp.float32)
        for d in range(_K):
            cold = jnp.full((_L,), d, jnp.int32)
            u = plsc.load_gather(urows, [row, cold])
            v = plsc.load_gather(vrows, [row, cold])
            acc = acc + u * v
        outv[pl.ds(g * _L, _L)] = 1.0 / (1.0 + jnp.exp(-acc))
        return 0

    lax.fori_loop(0, _B // _NW // _L, group, 0)
    pltpu.sync_copy(outv, out_hbm.at[pl.ds(base, _B // _NW)])


def _mesh():
    return plsc.VectorSubcoreMesh(core_axis_name="c", subcore_axis_name="s")


@jax.jit
def kernel(x, W, H):
    x = x.astype(jnp.int32)
    xu = x[:, 0]
    xv = x[:, 1]
    wtail = W[_TAIL0:, :].reshape(-1)
    htail = H[_TAIL0:, :].reshape(-1)

    nslot = _NW * _SLOTS
    k1 = pl.kernel(
        _k1_body,
        out_type=(jax.ShapeDtypeStruct((nslot * _K,), jnp.float32),
                  jax.ShapeDtypeStruct((nslot * _K,), jnp.float32),
                  jax.ShapeDtypeStruct((nslot,), jnp.int32),
                  jax.ShapeDtypeStruct((nslot,), jnp.int32)),
        mesh=_mesh(),
        scratch_types=[
            pltpu.VMEM((_B // 2,), jnp.int32),
            pltpu.VMEM((_B // 2,), jnp.int32),
            pltpu.VMEM((1024,), jnp.int32),
            pltpu.VMEM((1024,), jnp.int32),
            pltpu.VMEM((1024,), jnp.int32),
            pltpu.VMEM((1024,), jnp.int32),
            pltpu.VMEM((_SLOTS,), jnp.int32),
            pltpu.VMEM((_SLOTS,), jnp.int32),
            pltpu.VMEM((_SLOTS,), jnp.int32),
            pltpu.VMEM((_SLOTS,), jnp.int32),
            pltpu.VMEM((_NB,), jnp.int32),
            pltpu.VMEM((_NB,), jnp.int32),
            pltpu.VMEM((_K, _CH_R), jnp.float32),
            pltpu.VMEM((_K, _CH_R), jnp.float32),
            pltpu.VMEM((_K, _CH_R), jnp.float32),
            pltpu.VMEM((_K, _CH_R), jnp.float32),
            pltpu.VMEM((_K, 512), jnp.float32),
            pltpu.VMEM((_K, 512), jnp.float32),
            pltpu.VMEM((1024,), jnp.float32),
            pltpu.VMEM((1024,), jnp.float32),
            pltpu.VMEM((_CAP * _L,), jnp.float32),
            pltpu.VMEM((_CAP * _L,), jnp.float32),
            pltpu.VMEM((_CAP * _L,), jnp.float32),
            pltpu.VMEM((_CAP * _L,), jnp.float32),
            pltpu.SemaphoreType.DMA,
            pltpu.SemaphoreType.DMA,
            pltpu.SemaphoreType.DMA,
            pltpu.SemaphoreType.DMA,
            pltpu.SemaphoreType.DMA,
            pltpu.SemaphoreType.DMA,
        ],
        compiler_params=pltpu.CompilerParams(
            needs_layout_passes=False, use_tc_tiling_on_sc=True),
    )
    uf, vf, up, vp = k1(xu, xv, W.T, H.T, wtail, htail)

    k2 = pl.kernel(
        _k2_body,
        out_type=(jax.ShapeDtypeStruct((_B, _K), jnp.float32),
                  jax.ShapeDtypeStruct((_B, _K), jnp.float32)),
        mesh=_mesh(),
        scratch_types=[
            pltpu.VMEM((_SLOTS, _K), jnp.float32),
            pltpu.VMEM((_SLOTS, _K), jnp.float32),
            pltpu.VMEM((_SLOTS // 128, 128), jnp.int32),
            pltpu.VMEM((_SLOTS // 128, 128), jnp.int32),
            pltpu.SemaphoreType.DMA,
        ],
        compiler_params=pltpu.CompilerParams(
            needs_layout_passes=False, use_tc_tiling_on_sc=False),
    )
    us, vs = k2(uf.reshape(nslot, _K), vf.reshape(nslot, _K), up, vp)

    k3 = pl.kernel(
        _k3_body,
        out_type=jax.ShapeDtypeStruct((_B,), jnp.float32),
        mesh=_mesh(),
        scratch_types=[
            pltpu.VMEM((_B // _NW, _K), jnp.float32),
            pltpu.VMEM((_B // _NW, _K), jnp.float32),
            pltpu.VMEM((_B // _NW,), jnp.float32),
        ],
        compiler_params=pltpu.CompilerParams(
            needs_layout_passes=False, use_tc_tiling_on_sc=False),
    )
    return k3(us, vs)


# speedup vs baseline: 1.0615x; 1.0615x over previous
import functools

import jax
import jax.numpy as jnp
from jax import lax
from jax.experimental import pallas as pl
from jax.experimental.pallas import tpu as pltpu
from jax.experimental.pallas import tpu_sc as plsc

_B = 16384
_K = 16
_NC = 2
_NS = 16
_NW = _NC * _NS
_L = 16
_CH_R = 1024
_NB = 32
_CAP = 64
_SLOTS = _NB * _CAP
_ROW_LO = 262144
_FULL_END = 999424
_TAIL0 = 999936
_NU = 1000000


def _wid():
    return lax.axis_index("s") * _NC + lax.axis_index("c")


def _popcnt(m):
    pc = plsc.all_reduce_population_count(m)
    return lax.squeeze(lax.slice(pc, (0,), (1,)), (0,))


def _k1_body(xu_hbm, xv_hbm, wt_hbm, ht_hbm, wtail_hbm, htail_hbm,
             uflat_hbm, vflat_hbm, uposl_hbm, vposl_hbm,
             xlu, xlv, wlu_val, wlu_pos, wlv_val, wlv_pos,
             bu_u, bu_p, bv_u, bv_p, cnts_u, cnts_v,
             sw0, sw1, sh0, sh1, ew, eh, tw, th, st, stu1, stv0, stv1,
             semw0, semw1, semh0, semh1, semou, semov):
    wid = _wid()
    iota = lax.iota(jnp.int32, _L)
    lo = jnp.where(wid < 8, wid * 32768, _ROW_LO + (wid - 8) * 30720)
    nch = jnp.where(wid < 8, 32, 30)
    hi = jnp.where(wid == _NW - 1, _NU, lo + nch * _CH_R)
    lo_off = pl.multiple_of(lo, 128)

    pltpu.async_copy(wt_hbm.at[:, pl.ds(lo_off, _CH_R)], sw0, semw0)
    pltpu.async_copy(ht_hbm.at[:, pl.ds(lo_off, _CH_R)], sh0, semh0)
    off1 = pl.multiple_of(lo + _CH_R, 128)
    pltpu.async_copy(wt_hbm.at[:, pl.ds(off1, _CH_R)], sw1, semw1)
    pltpu.async_copy(ht_hbm.at[:, pl.ds(off1, _CH_R)], sh1, semh1)

    @pl.when(wid == _NW - 1)
    def _():
        pltpu.sync_copy(wt_hbm.at[:, pl.ds(_FULL_END, 512)], ew)
        pltpu.sync_copy(ht_hbm.at[:, pl.ds(_FULL_END, 512)], eh)
        pltpu.sync_copy(wtail_hbm, tw)
        pltpu.sync_copy(htail_hbm, th)

    neg1 = jnp.full((_L,), -1, jnp.int32)

    def init(i, _):
        bu_p[pl.ds(i * _L, _L)] = neg1
        bv_p[pl.ds(i * _L, _L)] = neg1
        return 0

    lax.fori_loop(0, _SLOTS // _L, init, 0)

    half_n = _B // 2

    def scan_half(p, carry):
        pltpu.sync_copy(xu_hbm.at[pl.ds(p * half_n, half_n)], xlu)
        pltpu.sync_copy(xv_hbm.at[pl.ds(p * half_n, half_n)], xlv)
        posb = p * half_n

        def scan(t, c2):
            cu, cv = c2
            u = xlu[pl.ds(t * _L, _L)]
            v = xlv[pl.ds(t * _L, _L)]
            mu = (u >= lo) & (u < hi)
            mv = (v >= lo) & (v < hi)
            pos = jnp.full((_L,), posb + t * _L, jnp.int32) + iota
            plsc.store_compressed(wlu_val.at[pl.ds(cu, _L)], u, mask=mu)
            plsc.store_compressed(wlu_pos.at[pl.ds(cu, _L)], pos, mask=mu)
            plsc.store_compressed(wlv_val.at[pl.ds(cv, _L)], v, mask=mv)
            plsc.store_compressed(wlv_pos.at[pl.ds(cv, _L)], pos, mask=mv)
            return (cu + _popcnt(mu), cv + _popcnt(mv))

        return lax.fori_loop(0, half_n // _L, scan, carry)

    nu, nv = lax.fori_loop(0, 2, scan_half, (jnp.int32(0), jnp.int32(0)))

    def distribute(n, wl_val, wl_pos, wb_u, wb_p, cnts_ref):
        def dist(q, cnts):
            val = wl_val[pl.ds(q * _L, _L)]
            pos = wl_pos[pl.ds(q * _L, _L)]
            valid = (jnp.full((_L,), q * _L, jnp.int32) + iota) < n
            bkt = lax.shift_right_logical(val - lo, 10)
            new = []
            for b in range(_NB):
                m = (bkt == b) & valid
                cb = cnts[b]
                plsc.store_compressed(
                    wb_u.at[pl.ds(b * _CAP + cb, _L)], val, mask=m)
                plsc.store_compressed(
                    wb_p.at[pl.ds(b * _CAP + cb, _L)], pos, mask=m)
                new.append(cb + _popcnt(m))
            return tuple(new)

        cnts = lax.fori_loop(0, (n + _L - 1) // _L, dist,
                             tuple(jnp.int32(0) for _ in range(_NB)))
        for half in range(2):
            acc = jnp.zeros((_L,), jnp.int32)
            for j in range(_L):
                acc = jnp.where(iota == j,
                                jnp.full((_L,), cnts[half * _L + j]), acc)
            cnts_ref[pl.ds(half * _L, _L)] = acc

    distribute(nu, wlu_val, wlu_pos, bu_u, bu_p, cnts_u)
    distribute(nv, wlv_val, wlv_pos, bv_u, bv_p, cnts_v)
    pltpu.sync_copy(bu_p, uposl_hbm.at[pl.ds(wid * _SLOTS, _SLOTS)])
    pltpu.sync_copy(bv_p, vposl_hbm.at[pl.ds(wid * _SLOTS, _SLOTS)])

    def gather_bucket(c, wb_u, cnts_ref, buf, stb, dst_hbm, sem_out, drain):
        cntc = jnp.max(plsc.load_gather(
            cnts_ref, [jnp.full((_L,), c, jnp.int32)]))
        cbase = lo + c * _CH_R

        @pl.when(drain)
        def _():
            pltpu.make_async_copy(
                stb, dst_hbm.at[pl.ds(0, _CAP * _L)], sem_out).wait()

        def grow(j, _):
            uj = plsc.load_gather(
                wb_u, [jnp.full((_L,), c * _CAP + j, jnp.int32)])
            ul = (uj - cbase) & (_CH_R - 1)
            stb[pl.ds(j * _L, _L)] = plsc.load_gather(buf, [iota, ul])
            return 0

        lax.fori_loop(0, cntc, grow, 0)
        dst = pl.multiple_of(wid * _SLOTS * _L + c * _CAP * _L, 512)
        pltpu.async_copy(stb, dst_hbm.at[pl.ds(dst, _CAP * _L)], sem_out)

    swb = (sw0, sw1)
    shb = (sh0, sh1)
    semw = (semw0, semw1)
    semh = (semh0, semh1)
    stu = (st, stu1)
    stv = (stv0, stv1)

    def pair(qp, _):
        for par in range(2):
            c = qp * 2 + par

            @pl.when(c < nch)
            def _():
                off = pl.multiple_of(lo + (c + 2) * _CH_R, 128)
                pltpu.make_async_copy(
                    wt_hbm.at[:, pl.ds(0, _CH_R)], swb[par], semw[par]).wait()
                gather_bucket(c, bu_u, cnts_u, swb[par], stu[par],
                              uflat_hbm, semou, c >= 2)

                @pl.when(c + 2 < nch)
                def _():
                    pltpu.async_copy(
                        wt_hbm.at[:, pl.ds(off, _CH_R)], swb[par], semw[par])

                pltpu.make_async_copy(
                    ht_hbm.at[:, pl.ds(0, _CH_R)], shb[par], semh[par]).wait()
                gather_bucket(c, bv_u, cnts_v, shb[par], stv[par],
                              vflat_hbm, semov, c >= 2)

                @pl.when(c + 2 < nch)
                def _():
                    pltpu.async_copy(
                        ht_hbm.at[:, pl.ds(off, _CH_R)], shb[par], semh[par])
        return 0

    lax.fori_loop(0, _NB // 2, pair, 0)

    for sem_out, dst_hbm in ((semou, uflat_hbm), (semov, vflat_hbm)):
        for _ in range(2):
            pltpu.make_async_copy(
                stu[0], dst_hbm.at[pl.ds(0, _CAP * _L)], sem_out).wait()

    @pl.when(wid == _NW - 1)
    def _():
        c = 30

        def tail_bucket(wb_u, cnts_ref, ebuf, tbuf, dst_hbm):
            cntc = jnp.max(plsc.load_gather(
                cnts_ref, [jnp.full((_L,), c, jnp.int32)]))

            def grow(j, _):
                uj = plsc.load_gather(
                    wb_u, [jnp.full((_L,), c * _CAP + j, jnp.int32)])
                ul = uj - _FULL_END
                in_slab = ul < 512
                row_a = plsc.load_gather(ebuf, [iota, ul & 511])
                row_b = plsc.load_gather(
                    tbuf, [((ul - 512) & 63) * _L + iota])
                st[pl.ds(j * _L, _L)] = jnp.where(in_slab, row_a, row_b)
                return 0

            lax.fori_loop(0, cntc, grow, 0)
            dst = pl.multiple_of(wid * _SLOTS * _L + c * _CAP * _L, 512)
            pltpu.sync_copy(st, dst_hbm.at[pl.ds(dst, _CAP * _L)])

        tail_bucket(bu_u, cnts_u, ew, tw, uflat_hbm)
        tail_bucket(bv_u, cnts_v, eh, th, vflat_hbm)


def _k2_body(uflat_hbm, vflat_hbm, uposl_hbm, vposl_hbm,
             usort_hbm, vsort_hbm, rstage_u, rstage_v, idx2_u, idx2_v, sem):
    wid = _wid()
    base = wid * _SLOTS
    nchunk = _SLOTS // 128
    stages = [
        pltpu.async_copy(uflat_hbm.at[pl.ds(base, _SLOTS), :], rstage_u, sem),
        pltpu.async_copy(vflat_hbm.at[pl.ds(base, _SLOTS), :], rstage_v, sem),
    ]
    for c in range(nchunk):
        stages.append(pltpu.async_copy(
            uposl_hbm.at[pl.ds(base + c * 128, 128)], idx2_u.at[c], sem))
        stages.append(pltpu.async_copy(
            vposl_hbm.at[pl.ds(base + c * 128, 128)], idx2_v.at[c], sem))
    for cp in stages:
        cp.wait()
    copies = []
    for c in range(nchunk):
        copies.append(pltpu.async_copy(
            rstage_u.at[pl.ds(c * 128, 128), :],
            usort_hbm.at[plsc.Indices(idx2_u.at[c], ignored_value=-1)],
            sem))
        copies.append(pltpu.async_copy(
            rstage_v.at[pl.ds(c * 128, 128), :],
            vsort_hbm.at[plsc.Indices(idx2_v.at[c], ignored_value=-1)],
            sem))
    for cp in copies:
        cp.wait()


def _k3_body(usort_hbm, vsort_hbm, out_hbm, urows, vrows, outv):
    wid = _wid()
    base = wid * (_B // _NW)
    iota = lax.iota(jnp.int32, _L)
    pltpu.sync_copy(usort_hbm.at[pl.ds(base, _B // _NW), :], urows)
    pltpu.sync_copy(vsort_hbm.at[pl.ds(base, _B // _NW), :], vrows)

    def group(g, _):
        row = jnp.full((_L,), g * _L, jnp.int32) + iota
        acc = jnp.zeros((_L,), jnp.float32)
        for d in range(_K):
            cold = jnp.full((_L,), d, jnp.int32)
            u = plsc.load_gather(urows, [row, cold])
            v = plsc.load_gather(vrows, [row, cold])
            acc = acc + u * v
        outv[pl.ds(g * _L, _L)] = 1.0 / (1.0 + jnp.exp(-acc))
        return 0

    lax.fori_loop(0, _B // _NW // _L, group, 0)
    pltpu.sync_copy(outv, out_hbm.at[pl.ds(base, _B // _NW)])


def _mesh():
    return plsc.VectorSubcoreMesh(core_axis_name="c", subcore_axis_name="s")


@jax.jit
def kernel(x, W, H):
    x = x.astype(jnp.int32)
    xu = x[:, 0]
    xv = x[:, 1]
    wtail = W[_TAIL0:, :].reshape(-1)
    htail = H[_TAIL0:, :].reshape(-1)

    nslot = _NW * _SLOTS
    k1 = pl.kernel(
        _k1_body,
        out_type=(jax.ShapeDtypeStruct((nslot * _K,), jnp.float32),
                  jax.ShapeDtypeStruct((nslot * _K,), jnp.float32),
                  jax.ShapeDtypeStruct((nslot,), jnp.int32),
                  jax.ShapeDtypeStruct((nslot,), jnp.int32)),
        mesh=_mesh(),
        scratch_types=[
            pltpu.VMEM((_B // 2,), jnp.int32),
            pltpu.VMEM((_B // 2,), jnp.int32),
            pltpu.VMEM((1024,), jnp.int32),
            pltpu.VMEM((1024,), jnp.int32),
            pltpu.VMEM((1024,), jnp.int32),
            pltpu.VMEM((1024,), jnp.int32),
            pltpu.VMEM((_SLOTS,), jnp.int32),
            pltpu.VMEM((_SLOTS,), jnp.int32),
            pltpu.VMEM((_SLOTS,), jnp.int32),
            pltpu.VMEM((_SLOTS,), jnp.int32),
            pltpu.VMEM((_NB,), jnp.int32),
            pltpu.VMEM((_NB,), jnp.int32),
            pltpu.VMEM((_K, _CH_R), jnp.float32),
            pltpu.VMEM((_K, _CH_R), jnp.float32),
            pltpu.VMEM((_K, _CH_R), jnp.float32),
            pltpu.VMEM((_K, _CH_R), jnp.float32),
            pltpu.VMEM((_K, 512), jnp.float32),
            pltpu.VMEM((_K, 512), jnp.float32),
            pltpu.VMEM((1024,), jnp.float32),
            pltpu.VMEM((1024,), jnp.float32),
            pltpu.VMEM((_CAP * _L,), jnp.float32),
            pltpu.VMEM((_CAP * _L,), jnp.float32),
            pltpu.VMEM((_CAP * _L,), jnp.float32),
            pltpu.VMEM((_CAP * _L,), jnp.float32),
            pltpu.SemaphoreType.DMA,
            pltpu.SemaphoreType.DMA,
            pltpu.SemaphoreType.DMA,
            pltpu.SemaphoreType.DMA,
            pltpu.SemaphoreType.DMA,
            pltpu.SemaphoreType.DMA,
        ],
        compiler_params=pltpu.CompilerParams(
            needs_layout_passes=False, use_tc_tiling_on_sc=True),
    )
    uf, vf, up, vp = k1(xu, xv, W.T, H.T, wtail, htail)

    k2 = pl.kernel(
        _k2_body,
        out_type=(jax.ShapeDtypeStruct((_B, _K), jnp.float32),
                  jax.ShapeDtypeStruct((_B, _K), jnp.float32)),
        mesh=_mesh(),
        scratch_types=[
            pltpu.VMEM((_SLOTS, _K), jnp.float32),
            pltpu.VMEM((_SLOTS, _K), jnp.float32),
            pltpu.VMEM((_SLOTS // 128, 128), jnp.int32),
            pltpu.VMEM((_SLOTS // 128, 128), jnp.int32),
            pltpu.SemaphoreType.DMA,
        ],
        compiler_params=pltpu.CompilerParams(
            needs_layout_passes=False, use_tc_tiling_on_sc=False),
    )
    us, vs = k2(uf.reshape(nslot, _K), vf.reshape(nslot, _K), up, vp)

    k3 = pl.kernel(
        _k3_body,
        out_type=jax.ShapeDtypeStruct((_B,), jnp.float32),
        mesh=_mesh(),
        scratch_types=[
            pltpu.VMEM((_B // _NW, _K), jnp.float32),
            pltpu.VMEM((_B // _NW, _K), jnp.float32),
            pltpu.VMEM((_B // _NW,), jnp.float32),
        ],
        compiler_params=pltpu.CompilerParams(
            needs_layout_passes=False, use_tc_tiling_on_sc=False),
    )
    return k3(us, vs)


# speedup vs baseline: 1.0642x; 1.0026x over previous
import jax
import jax.numpy as jnp
from jax import lax
from jax.experimental import pallas as pl
from jax.experimental.pallas import tpu as pltpu
from jax.experimental.pallas import tpu_sc as plsc

_B = 16384
_K = 16
_NC = 2
_NS = 16
_NW = _NC * _NS
_L = 16
_CH_R = 1024
_NB = 32
_CAP = 64
_SLOTS = _NB * _CAP
_ROW_LO = 262144
_FULL_END = 999424
_TAIL0 = 999936
_NU = 1000000


def _wid():
    return lax.axis_index("s") * _NC + lax.axis_index("c")


def _popcnt(m):
    pc = plsc.all_reduce_population_count(m)
    return lax.squeeze(lax.slice(pc, (0,), (1,)), (0,))


def _k1_body(xu_hbm, xv_hbm, wt_hbm, ht_hbm, wtail_hbm, htail_hbm,
             uflat_hbm, vflat_hbm, uposl_hbm, vposl_hbm,
             xlu, xlv, wlu_val, wlu_pos, wlv_val, wlv_pos,
             bu_u, bu_p, bv_u, bv_p, cnts_u, cnts_v,
             sw0, sw1, sh0, sh1, ew, eh, tw, th, st, stu1, stv0, stv1,
             semw0, semw1, semh0, semh1, semou, semov):
    wid = _wid()
    iota = lax.iota(jnp.int32, _L)
    lo = jnp.where(wid < 8, wid * 32768, _ROW_LO + (wid - 8) * 30720)
    nch = jnp.where(wid < 8, 32, 30)
    hi = jnp.where(wid == _NW - 1, _NU, lo + nch * _CH_R)
    lo_off = pl.multiple_of(lo, 128)

    pltpu.async_copy(wt_hbm.at[:, pl.ds(lo_off, _CH_R)], sw0, semw0)
    pltpu.async_copy(ht_hbm.at[:, pl.ds(lo_off, _CH_R)], sh0, semh0)
    off1 = pl.multiple_of(lo + _CH_R, 128)
    pltpu.async_copy(wt_hbm.at[:, pl.ds(off1, _CH_R)], sw1, semw1)
    pltpu.async_copy(ht_hbm.at[:, pl.ds(off1, _CH_R)], sh1, semh1)

    @pl.when(wid == _NW - 1)
    def _():
        pltpu.sync_copy(wt_hbm.at[:, pl.ds(_FULL_END, 512)], ew)
        pltpu.sync_copy(ht_hbm.at[:, pl.ds(_FULL_END, 512)], eh)
        pltpu.sync_copy(wtail_hbm, tw)
        pltpu.sync_copy(htail_hbm, th)

    neg1 = jnp.full((_L,), -1, jnp.int32)

    def init(i, _):
        bu_p[pl.ds(i * _L, _L)] = neg1
        bv_p[pl.ds(i * _L, _L)] = neg1
        return 0

    lax.fori_loop(0, _SLOTS // _L, init, 0)

    half_n = _B // 2

    def scan_half(p, carry):
        pltpu.sync_copy(xu_hbm.at[pl.ds(p * half_n, half_n)], xlu)
        pltpu.sync_copy(xv_hbm.at[pl.ds(p * half_n, half_n)], xlv)
        posb = p * half_n

        def scan(t, c2):
            cu, cv = c2
            u = xlu[pl.ds(t * _L, _L)]
            v = xlv[pl.ds(t * _L, _L)]
            mu = (u >= lo) & (u < hi)
            mv = (v >= lo) & (v < hi)
            pos = jnp.full((_L,), posb + t * _L, jnp.int32) + iota
            plsc.store_compressed(wlu_val.at[pl.ds(cu, _L)], u, mask=mu)
            plsc.store_compressed(wlu_pos.at[pl.ds(cu, _L)], pos, mask=mu)
            plsc.store_compressed(wlv_val.at[pl.ds(cv, _L)], v, mask=mv)
            plsc.store_compressed(wlv_pos.at[pl.ds(cv, _L)], pos, mask=mv)
            return (cu + _popcnt(mu), cv + _popcnt(mv))

        return lax.fori_loop(0, half_n // _L, scan, carry)

    nu, nv = lax.fori_loop(0, 2, scan_half, (jnp.int32(0), jnp.int32(0)))

    def distribute(n, wl_val, wl_pos, wb_u, wb_p, cnts_ref):
        def dist(q, cnts):
            val = wl_val[pl.ds(q * _L, _L)]
            pos = wl_pos[pl.ds(q * _L, _L)]
            valid = (jnp.full((_L,), q * _L, jnp.int32) + iota) < n
            bkt = lax.shift_right_logical(val - lo, 10)
            new = []
            for b in range(_NB):
                m = (bkt == b) & valid
                cb = cnts[b]
                plsc.store_compressed(
                    wb_u.at[pl.ds(b * _CAP + cb, _L)], val, mask=m)
                plsc.store_compressed(
                    wb_p.at[pl.ds(b * _CAP + cb, _L)], pos, mask=m)
                new.append(cb + _popcnt(m))
            return tuple(new)

        cnts = lax.fori_loop(0, (n + _L - 1) // _L, dist,
                             tuple(jnp.int32(0) for _ in range(_NB)))
        for half in range(2):
            acc = jnp.zeros((_L,), jnp.int32)
            for j in range(_L):
                acc = jnp.where(iota == j,
                                jnp.full((_L,), cnts[half * _L + j]), acc)
            cnts_ref[pl.ds(half * _L, _L)] = acc

    distribute(nu, wlu_val, wlu_pos, bu_u, bu_p, cnts_u)
    distribute(nv, wlv_val, wlv_pos, bv_u, bv_p, cnts_v)
    pltpu.sync_copy(bu_p, uposl_hbm.at[pl.ds(wid * _SLOTS, _SLOTS)])
    pltpu.sync_copy(bv_p, vposl_hbm.at[pl.ds(wid * _SLOTS, _SLOTS)])

    def gather_bucket(c, wb_u, cnts_ref, buf, stb, dst_hbm, sem_out, drain):
        cntc = jnp.max(plsc.load_gather(
            cnts_ref, [jnp.full((_L,), c, jnp.int32)]))
        cbase = lo + c * _CH_R

        @pl.when(drain)
        def _():
            pltpu.make_async_copy(
                stb, dst_hbm.at[pl.ds(0, _CAP * _L)], sem_out).wait()

        def grow(j, _):
            uj = plsc.load_gather(
                wb_u, [jnp.full((_L,), c * _CAP + j, jnp.int32)])
            ul = (uj - cbase) & (_CH_R - 1)
            stb[pl.ds(j * _L, _L)] = plsc.load_gather(buf, [iota, ul])
            return 0

        lax.fori_loop(0, cntc, grow, 0)
        dst = pl.multiple_of(wid * _SLOTS * _L + c * _CAP * _L, 512)
        pltpu.async_copy(stb, dst_hbm.at[pl.ds(dst, _CAP * _L)], sem_out)

    swb = (sw0, sw1)
    shb = (sh0, sh1)
    semw = (semw0, semw1)
    semh = (semh0, semh1)
    stu = (st, stu1)
    stv = (stv0, stv1)

    def pair(qp, _):
        for par in range(2):
            c = qp * 2 + par

            @pl.when(c < nch)
            def _():
                off = pl.multiple_of(lo + (c + 2) * _CH_R, 128)
                pltpu.make_async_copy(
                    wt_hbm.at[:, pl.ds(0, _CH_R)], swb[par], semw[par]).wait()
                gather_bucket(c, bu_u, cnts_u, swb[par], stu[par],
                              uflat_hbm, semou, c >= 2)

                @pl.when(c + 2 < nch)
                def _():
                    pltpu.async_copy(
                        wt_hbm.at[:, pl.ds(off, _CH_R)], swb[par], semw[par])

                pltpu.make_async_copy(
                    ht_hbm.at[:, pl.ds(0, _CH_R)], shb[par], semh[par]).wait()
                gather_bucket(c, bv_u, cnts_v, shb[par], stv[par],
                              vflat_hbm, semov, c >= 2)

                @pl.when(c + 2 < nch)
                def _():
                    pltpu.async_copy(
                        ht_hbm.at[:, pl.ds(off, _CH_R)], shb[par], semh[par])
        return 0

    lax.fori_loop(0, _NB // 2, pair, 0)

    for sem_out, dst_hbm in ((semou, uflat_hbm), (semov, vflat_hbm)):
        for _ in range(2):
            pltpu.make_async_copy(
                stu[0], dst_hbm.at[pl.ds(0, _CAP * _L)], sem_out).wait()

    @pl.when(wid == _NW - 1)
    def _():
        c = 30

        def tail_bucket(wb_u, cnts_ref, ebuf, tbuf, dst_hbm):
            cntc = jnp.max(plsc.load_gather(
                cnts_ref, [jnp.full((_L,), c, jnp.int32)]))

            def grow(j, _):
                uj = plsc.load_gather(
                    wb_u, [jnp.full((_L,), c * _CAP + j, jnp.int32)])
                ul = uj - _FULL_END
                in_slab = ul < 512
                row_a = plsc.load_gather(ebuf, [iota, ul & 511])
                row_b = plsc.load_gather(
                    tbuf, [((ul - 512) & 63) * _L + iota])
                st[pl.ds(j * _L, _L)] = jnp.where(in_slab, row_a, row_b)
                return 0

            lax.fori_loop(0, cntc, grow, 0)
            dst = pl.multiple_of(wid * _SLOTS * _L + c * _CAP * _L, 512)
            pltpu.sync_copy(st, dst_hbm.at[pl.ds(dst, _CAP * _L)])

        tail_bucket(bu_u, cnts_u, ew, tw, uflat_hbm)
        tail_bucket(bv_u, cnts_v, eh, th, vflat_hbm)


def _k2_body(uflat_hbm, vflat_hbm, uposl_hbm, vposl_hbm,
             usort_hbm, vsort_hbm, rstage_u, rstage_v, idx2_u, idx2_v, sem):
    wid = _wid()
    base = wid * _SLOTS
    nchunk = _SLOTS // 128
    stages = [
        pltpu.async_copy(uflat_hbm.at[pl.ds(base, _SLOTS), :], rstage_u, sem),
        pltpu.async_copy(vflat_hbm.at[pl.ds(base, _SLOTS), :], rstage_v, sem),
    ]
    for c in range(nchunk):
        stages.append(pltpu.async_copy(
            uposl_hbm.at[pl.ds(base + c * 128, 128)], idx2_u.at[c], sem))
        stages.append(pltpu.async_copy(
            vposl_hbm.at[pl.ds(base + c * 128, 128)], idx2_v.at[c], sem))
    for cp in stages:
        cp.wait()
    copies = []
    for c in range(nchunk):
        copies.append(pltpu.async_copy(
            rstage_u.at[pl.ds(c * 128, 128), :],
            usort_hbm.at[plsc.Indices(idx2_u.at[c], ignored_value=-1)],
            sem))
        copies.append(pltpu.async_copy(
            rstage_v.at[pl.ds(c * 128, 128), :],
            vsort_hbm.at[plsc.Indices(idx2_v.at[c], ignored_value=-1)],
            sem))
    for cp in copies:
        cp.wait()


def _k3_body(usort_hbm, vsort_hbm, out_hbm, urows, vrows, outv):
    wid = _wid()
    base = wid * (_B // _NW)
    iota = lax.iota(jnp.int32, _L)
    pltpu.sync_copy(usort_hbm.at[pl.ds(base, _B // _NW), :], urows)
    pltpu.sync_copy(vsort_hbm.at[pl.ds(base, _B // _NW), :], vrows)

    def group(g, _):
        row = jnp.full((_L,), g * _L, jnp.int32) + iota
        acc = jnp.zeros((_L,), jnp.float32)
        for d in range(_K):
            cold = jnp.full((_L,), d, jnp.int32)
            u = plsc.load_gather(urows, [row, cold])
            v = plsc.load_gather(vrows, [row, cold])
            acc = acc + u * v
        outv[pl.ds(g * _L, _L)] = 1.0 / (1.0 + jnp.exp(-acc))
        return 0

    lax.fori_loop(0, _B // _NW // _L, group, 0)
    pltpu.sync_copy(outv, out_hbm.at[pl.ds(base, _B // _NW)])


def _mesh():
    return plsc.VectorSubcoreMesh(core_axis_name="c", subcore_axis_name="s")


@jax.jit
def kernel(x, W, H):
    x = x.astype(jnp.int32)
    xu = x[:, 0]
    xv = x[:, 1]
    wtail = W[_TAIL0:, :].reshape(-1)
    htail = H[_TAIL0:, :].reshape(-1)

    nslot = _NW * _SLOTS
    k1 = pl.kernel(
        _k1_body,
        out_type=(jax.ShapeDtypeStruct((nslot * _K,), jnp.float32),
                  jax.ShapeDtypeStruct((nslot * _K,), jnp.float32),
                  jax.ShapeDtypeStruct((nslot,), jnp.int32),
                  jax.ShapeDtypeStruct((nslot,), jnp.int32)),
        mesh=_mesh(),
        scratch_types=[
            pltpu.VMEM((_B // 2,), jnp.int32),
            pltpu.VMEM((_B // 2,), jnp.int32),
            pltpu.VMEM((1024,), jnp.int32),
            pltpu.VMEM((1024,), jnp.int32),
            pltpu.VMEM((1024,), jnp.int32),
            pltpu.VMEM((1024,), jnp.int32),
            pltpu.VMEM((_SLOTS,), jnp.int32),
            pltpu.VMEM((_SLOTS,), jnp.int32),
            pltpu.VMEM((_SLOTS,), jnp.int32),
            pltpu.VMEM((_SLOTS,), jnp.int32),
            pltpu.VMEM((_NB,), jnp.int32),
            pltpu.VMEM((_NB,), jnp.int32),
            pltpu.VMEM((_K, _CH_R), jnp.float32),
            pltpu.VMEM((_K, _CH_R), jnp.float32),
            pltpu.VMEM((_K, _CH_R), jnp.float32),
            pltpu.VMEM((_K, _CH_R), jnp.float32),
            pltpu.VMEM((_K, 512), jnp.float32),
            pltpu.VMEM((_K, 512), jnp.float32),
            pltpu.VMEM((1024,), jnp.float32),
            pltpu.VMEM((1024,), jnp.float32),
            pltpu.VMEM((_CAP * _L,), jnp.float32),
            pltpu.VMEM((_CAP * _L,), jnp.float32),
            pltpu.VMEM((_CAP * _L,), jnp.float32),
            pltpu.VMEM((_CAP * _L,), jnp.float32),
            pltpu.SemaphoreType.DMA,
            pltpu.SemaphoreType.DMA,
            pltpu.SemaphoreType.DMA,
            pltpu.SemaphoreType.DMA,
            pltpu.SemaphoreType.DMA,
            pltpu.SemaphoreType.DMA,
        ],
        compiler_params=pltpu.CompilerParams(
            needs_layout_passes=False, use_tc_tiling_on_sc=True),
    )
    uf, vf, up, vp = k1(xu, xv, W.T, H.T, wtail, htail)

    k2 = pl.kernel(
        _k2_body,
        out_type=(jax.ShapeDtypeStruct((_B, _K), jnp.float32),
                  jax.ShapeDtypeStruct((_B, _K), jnp.float32)),
        mesh=_mesh(),
        scratch_types=[
            pltpu.VMEM((_SLOTS, _K), jnp.float32),
            pltpu.VMEM((_SLOTS, _K), jnp.float32),
            pltpu.VMEM((_SLOTS // 128, 128), jnp.int32),
            pltpu.VMEM((_SLOTS // 128, 128), jnp.int32),
            pltpu.SemaphoreType.DMA,
        ],
        compiler_params=pltpu.CompilerParams(
            needs_layout_passes=False, use_tc_tiling_on_sc=False),
    )
    us, vs = k2(uf.reshape(nslot, _K), vf.reshape(nslot, _K), up, vp)

    k3 = pl.kernel(
        _k3_body,
        out_type=jax.ShapeDtypeStruct((_B,), jnp.float32),
        mesh=_mesh(),
        scratch_types=[
            pltpu.VMEM((_B // _NW, _K), jnp.float32),
            pltpu.VMEM((_B // _NW, _K), jnp.float32),
            pltpu.VMEM((_B // _NW,), jnp.float32),
        ],
        compiler_params=pltpu.CompilerParams(
            needs_layout_passes=False, use_tc_tiling_on_sc=False),
    )
    return k3(us, vs)
